# baseline (speedup 1.0000x reference)
import numpy as np

import jax
import jax.numpy as jnp
from jax.experimental import pallas as pl
from jax.experimental.pallas import tpu as pltpu

N = 10000
H = 128
BM = 512
NB = (N + BM - 1) // BM
NPAD = NB * BM
QR = 4 * BM
NQ = NPAD // QR
BC = 1024
NC = NPAD // BC


def _pass1_body(adj_ref, comb_in_ref, w1_ref, s1_ref, w2_ref, s2_ref,
                h1_out_ref, part_out_ref, comb_ref, stage_ref):
    r = pl.program_id(0)

    a = adj_ref[...].astype(jnp.bfloat16)
    res = jnp.dot(a, comb_ref[pl.ds(0, N), :],
                  preferred_element_type=jnp.float32)
    pooled = res[:, :H]
    part_out_ref[...] = res[:, H:]

    t = jnp.maximum(
        jnp.dot(pooled, w1_ref[...], preferred_element_type=jnp.float32)
        + s1_ref[...], 0.0)
    h1b = jnp.maximum(
        jnp.dot(t, w2_ref[...], preferred_element_type=jnp.float32)
        + s2_ref[...], 0.0)
    row_ids = r * BM + jax.lax.broadcasted_iota(jnp.int32, (BM, H), 0)
    h1b = jnp.where(row_ids < N, h1b, 0.0).astype(jnp.bfloat16)

    h1_out_ref[...] = h1b

    @pl.when(r % 4 != 3)
    def _stage():
        s_off = pl.multiple_of((r % 4) * BM, 16)
        stage_ref[pl.ds(s_off, BM), :] = h1b

    @pl.when(r % 4 == 3)
    def _commit():
        q_off = pl.multiple_of((r - 3) * BM, 16)
        comb_ref[pl.ds(q_off, 3 * BM), pl.ds(H, H)] = stage_ref[...]
        comb_ref[pl.ds(q_off + 3 * BM, BM), pl.ds(H, H)] = h1b


def _const(shape):
    return pl.BlockSpec(shape, lambda i: (0,) * len(shape))


def _pass1_call(adj, h0, w1, s1, w2, s2):
    comb0 = jnp.zeros((NPAD, 2 * H), jnp.bfloat16)
    comb0 = jax.lax.dynamic_update_slice(comb0, h0, (0, 0))
    h1, part, _ = pl.pallas_call(
        _pass1_body,
        grid=(NB,),
        in_specs=[
            pl.BlockSpec((BM, N), lambda i: (i, 0)),
            _const((NPAD, 2 * H)),
            _const((H, H)),
            _const((1, H)),
            _const((H, H)),
            _const((1, H)),
        ],
        out_specs=[
            pl.BlockSpec((BM, H), lambda i: (i, 0)),
            pl.BlockSpec((BM, H), lambda i: (i, 0)),
            _const((NPAD, 2 * H)),
        ],
        out_shape=[
            jax.ShapeDtypeStruct((NPAD, H), jnp.bfloat16),
            jax.ShapeDtypeStruct((NPAD, H), jnp.float32),
            jax.ShapeDtypeStruct((NPAD, 2 * H), jnp.bfloat16),
        ],
        input_output_aliases={1: 2},
        scratch_shapes=[pltpu.VMEM((3 * BM, H), jnp.bfloat16)],
        compiler_params=pltpu.CompilerParams(
            dimension_semantics=("arbitrary",)),
    )(adj, comb0, w1, s1, w2, s2)
    return h1, part


def _tile_schedule():
    qs, cs, first, last = [], [], [], []
    for q in range(NQ):
        for j in range(2 * q, NC):
            qs.append(q)
            cs.append(j)
            first.append(1 if j == 2 * q else 0)
            last.append(1 if j == NC - 1 else 0)
    to = lambda x: jnp.asarray(np.array(x, dtype=np.int32))
    return to(qs), to(cs), to(first), to(last)


def _pass2_body(qs_ref, cs_ref, first_ref, last_ref,
                adj_ref, h1_ref, part_ref,
                w1_ref, s1_ref, w2_ref, s2_ref, wp_ref, bp_ref,
                out_ref, acc_ref):
    t = pl.program_id(0)

    @pl.when(first_ref[t] == 1)
    def _init():
        acc_ref[...] = part_ref[...]

    cidx = cs_ref[t]
    c_off = pl.multiple_of(cidx * BC, 16)
    rhs = h1_ref[pl.ds(c_off, BC), :]

    @pl.when(cidx < NC - 1)
    def _plain():
        acc_ref[...] = acc_ref[...] + jnp.dot(
            adj_ref[...].astype(jnp.bfloat16), rhs,
            preferred_element_type=jnp.float32)

    @pl.when(cidx == NC - 1)
    def _masked():
        col_ids = jax.lax.broadcasted_iota(jnp.int32, (QR, BC), 1)
        a = jnp.where(c_off + col_ids < N, adj_ref[...], 0.0)
        acc_ref[...] = acc_ref[...] + jnp.dot(
            a.astype(jnp.bfloat16), rhs,
            preferred_element_type=jnp.float32)

    @pl.when(last_ref[t] == 1)
    def _epilogue():
        tt = jnp.maximum(
            jnp.dot(acc_ref[...], w1_ref[...],
                    preferred_element_type=jnp.float32) + s1_ref[...], 0.0)
        h2 = jnp.maximum(
            jnp.dot(tt, w2_ref[...],
                    preferred_element_type=jnp.float32) + s2_ref[...], 0.0)
        out_ref[...] = (jnp.dot(h2, wp_ref[...],
                                preferred_element_type=jnp.float32)
                        + bp_ref[...])


def _pass2_call(adj, h1, part, w1, s1, w2, s2, wp, bp):
    qs, cs, first, last = _tile_schedule()
    ntiles = int(qs.shape[0])

    def _c(shape):
        return pl.BlockSpec(shape, lambda t, *s: (0,) * len(shape))

    grid_spec = pltpu.PrefetchScalarGridSpec(
        num_scalar_prefetch=4,
        grid=(ntiles,),
        in_specs=[
            pl.BlockSpec((QR, BC), lambda t, qs, cs, *s: (qs[t], cs[t])),
            _c((NPAD, H)),
            pl.BlockSpec((QR, H), lambda t, qs, *s: (qs[t], 0)),
            _c((H, H)),
            _c((1, H)),
            _c((H, H)),
            _c((1, H)),
            _c((H, 1)),
            _c((1, 1)),
        ],
        out_specs=pl.BlockSpec((QR, 1), lambda t, qs, *s: (qs[t], 0)),
        scratch_shapes=[pltpu.VMEM((QR, H), jnp.float32)],
    )
    return pl.pallas_call(
        _pass2_body,
        grid_spec=grid_spec,
        out_shape=jax.ShapeDtypeStruct((N, 1), jnp.float32),
        compiler_params=pltpu.CompilerParams(
            dimension_semantics=("arbitrary",)),
    )(qs, cs, first, last, adj, h1, part, w1, s1, w2, s2, wp, bp)


def _fold_bn(W1, b1, g1, be1, W2, b2, g, be):
    inv = 1.0 / jnp.sqrt(1.0 + 1e-5)
    sc1 = g1 * inv
    sc2 = g * inv
    w1 = W1 * sc1[None, :]
    s1 = (b1 * sc1 + be1)[None, :]
    w2 = W2 * sc2[None, :]
    s2 = (b2 * sc2 + be)[None, :]
    return w1, s1, w2, s2


def kernel(seq1, adj, W1_0, b1_0, g1_0, be1_0, W2_0, b2_0, g_0, be_0,
           W1_1, b1_1, g1_1, be1_1, W2_1, b2_1, g_1, be_1, Wp, bp):
    w1a, s1a, w2a, s2a = _fold_bn(W1_0, b1_0, g1_0, be1_0, W2_0, b2_0, g_0, be_0)
    w1b, s1b, w2b, s2b = _fold_bn(W1_1, b1_1, g1_1, be1_1, W2_1, b2_1, g_1, be_1)
    h0 = seq1.astype(jnp.bfloat16)
    h1, part = _pass1_call(adj, h0, w1a, s1a, w2a, s2a)
    return _pass2_call(adj, h1, part, w1b, s1b, w2b, s2b,
                       Wp, bp.reshape(1, 1))


# speedup vs baseline: 1.0101x; 1.0101x over previous
import numpy as np

import jax
import jax.numpy as jnp
from jax.experimental import pallas as pl
from jax.experimental.pallas import tpu as pltpu

N = 10000
H = 128
BM = 512
NB = (N + BM - 1) // BM
NPAD = NB * BM
QR = 4 * BM
NQ = NPAD // QR
BC = 2048
NC = NPAD // BC


def _pass1_body(adj_ref, comb_in_ref, w1_ref, s1_ref, w2_ref, s2_ref,
                h1_out_ref, part_out_ref, comb_ref, stage_ref):
    r = pl.program_id(0)

    a = adj_ref[...].astype(jnp.bfloat16)
    res = jnp.dot(a, comb_ref[pl.ds(0, N), :],
                  preferred_element_type=jnp.float32)
    pooled = res[:, :H]
    part_out_ref[...] = res[:, H:]

    t = jnp.maximum(
        jnp.dot(pooled, w1_ref[...], preferred_element_type=jnp.float32)
        + s1_ref[...], 0.0)
    h1b = jnp.maximum(
        jnp.dot(t, w2_ref[...], preferred_element_type=jnp.float32)
        + s2_ref[...], 0.0)
    row_ids = r * BM + jax.lax.broadcasted_iota(jnp.int32, (BM, H), 0)
    h1b = jnp.where(row_ids < N, h1b, 0.0).astype(jnp.bfloat16)

    h1_out_ref[...] = h1b

    @pl.when(r % 4 != 3)
    def _stage():
        s_off = pl.multiple_of((r % 4) * BM, 16)
        stage_ref[pl.ds(s_off, BM), :] = h1b

    @pl.when(r % 4 == 3)
    def _commit():
        q_off = pl.multiple_of((r - 3) * BM, 16)
        comb_ref[pl.ds(q_off, 3 * BM), pl.ds(H, H)] = stage_ref[...]
        comb_ref[pl.ds(q_off + 3 * BM, BM), pl.ds(H, H)] = h1b


def _const(shape):
    return pl.BlockSpec(shape, lambda i: (0,) * len(shape))


def _pass1_call(adj, h0, w1, s1, w2, s2):
    comb0 = jnp.zeros((NPAD, 2 * H), jnp.bfloat16)
    comb0 = jax.lax.dynamic_update_slice(comb0, h0, (0, 0))
    h1, part, _ = pl.pallas_call(
        _pass1_body,
        grid=(NB,),
        in_specs=[
            pl.BlockSpec((BM, N), lambda i: (i, 0)),
            _const((NPAD, 2 * H)),
            _const((H, H)),
            _const((1, H)),
            _const((H, H)),
            _const((1, H)),
        ],
        out_specs=[
            pl.BlockSpec((BM, H), lambda i: (i, 0)),
            pl.BlockSpec((BM, H), lambda i: (i, 0)),
            _const((NPAD, 2 * H)),
        ],
        out_shape=[
            jax.ShapeDtypeStruct((NPAD, H), jnp.bfloat16),
            jax.ShapeDtypeStruct((NPAD, H), jnp.float32),
            jax.ShapeDtypeStruct((NPAD, 2 * H), jnp.bfloat16),
        ],
        input_output_aliases={1: 2},
        scratch_shapes=[pltpu.VMEM((3 * BM, H), jnp.bfloat16)],
        compiler_params=pltpu.CompilerParams(
            dimension_semantics=("arbitrary",)),
    )(adj, comb0, w1, s1, w2, s2)
    return h1, part


def _tile_schedule():
    qs, cs, first, last = [], [], [], []
    for q in range(NQ):
        for j in range(q, NC):
            qs.append(q)
            cs.append(j)
            first.append(1 if j == q else 0)
            last.append(1 if j == NC - 1 else 0)
    to = lambda x: jnp.asarray(np.array(x, dtype=np.int32))
    return to(qs), to(cs), to(first), to(last)


def _pass2_body(qs_ref, cs_ref, first_ref, last_ref,
                adj_ref, h1_ref, part_ref,
                w1_ref, s1_ref, w2_ref, s2_ref, wp_ref, bp_ref,
                out_ref, acc_ref):
    t = pl.program_id(0)

    @pl.when(first_ref[t] == 1)
    def _init():
        acc_ref[...] = part_ref[...]

    cidx = cs_ref[t]
    c_off = pl.multiple_of(cidx * BC, 16)
    rhs = h1_ref[pl.ds(c_off, BC), :]

    @pl.when(cidx < NC - 1)
    def _plain():
        acc_ref[...] = acc_ref[...] + jnp.dot(
            adj_ref[...].astype(jnp.bfloat16), rhs,
            preferred_element_type=jnp.float32)

    @pl.when(cidx == NC - 1)
    def _masked():
        col_ids = jax.lax.broadcasted_iota(jnp.int32, (QR, BC), 1)
        a = jnp.where(c_off + col_ids < N, adj_ref[...], 0.0)
        acc_ref[...] = acc_ref[...] + jnp.dot(
            a.astype(jnp.bfloat16), rhs,
            preferred_element_type=jnp.float32)

    @pl.when(last_ref[t] == 1)
    def _epilogue():
        tt = jnp.maximum(
            jnp.dot(acc_ref[...], w1_ref[...],
                    preferred_element_type=jnp.float32) + s1_ref[...], 0.0)
        h2 = jnp.maximum(
            jnp.dot(tt, w2_ref[...],
                    preferred_element_type=jnp.float32) + s2_ref[...], 0.0)
        out_ref[...] = (jnp.dot(h2, wp_ref[...],
                                preferred_element_type=jnp.float32)
                        + bp_ref[...])


def _pass2_call(adj, h1, part, w1, s1, w2, s2, wp, bp):
    qs, cs, first, last = _tile_schedule()
    ntiles = int(qs.shape[0])

    def _c(shape):
        return pl.BlockSpec(shape, lambda t, *s: (0,) * len(shape))

    grid_spec = pltpu.PrefetchScalarGridSpec(
        num_scalar_prefetch=4,
        grid=(ntiles,),
        in_specs=[
            pl.BlockSpec((QR, BC), lambda t, qs, cs, *s: (qs[t], cs[t])),
            _c((NPAD, H)),
            pl.BlockSpec((QR, H), lambda t, qs, *s: (qs[t], 0)),
            _c((H, H)),
            _c((1, H)),
            _c((H, H)),
            _c((1, H)),
            _c((H, 1)),
            _c((1, 1)),
        ],
        out_specs=pl.BlockSpec((QR, 1), lambda t, qs, *s: (qs[t], 0)),
        scratch_shapes=[pltpu.VMEM((QR, H), jnp.float32)],
    )
    return pl.pallas_call(
        _pass2_body,
        grid_spec=grid_spec,
        out_shape=jax.ShapeDtypeStruct((N, 1), jnp.float32),
        compiler_params=pltpu.CompilerParams(
            dimension_semantics=("arbitrary",)),
    )(qs, cs, first, last, adj, h1, part, w1, s1, w2, s2, wp, bp)


def _fold_bn(W1, b1, g1, be1, W2, b2, g, be):
    inv = 1.0 / jnp.sqrt(1.0 + 1e-5)
    sc1 = g1 * inv
    sc2 = g * inv
    w1 = W1 * sc1[None, :]
    s1 = (b1 * sc1 + be1)[None, :]
    w2 = W2 * sc2[None, :]
    s2 = (b2 * sc2 + be)[None, :]
    return w1, s1, w2, s2


def kernel(seq1, adj, W1_0, b1_0, g1_0, be1_0, W2_0, b2_0, g_0, be_0,
           W1_1, b1_1, g1_1, be1_1, W2_1, b2_1, g_1, be_1, Wp, bp):
    w1a, s1a, w2a, s2a = _fold_bn(W1_0, b1_0, g1_0, be1_0, W2_0, b2_0, g_0, be_0)
    w1b, s1b, w2b, s2b = _fold_bn(W1_1, b1_1, g1_1, be1_1, W2_1, b2_1, g_1, be_1)
    h0 = seq1.astype(jnp.bfloat16)
    h1, part = _pass1_call(adj, h0, w1a, s1a, w2a, s2a)
    return _pass2_call(adj, h1, part, w1b, s1b, w2b, s2b,
                       Wp, bp.reshape(1, 1))
